# baseline (speedup 1.0000x reference)
import jax
import jax.numpy as jnp
from jax.experimental import pallas as pl

_NUM_BUCKETS = 1024.0
_BLOCK_C = 49152
_WGROUP = 4


def _lsh_block_kernel(xt_ref, p_ref, b_ref, o_ref):
    h = jax.lax.dot_general(
        p_ref[...], xt_ref[...],
        dimension_numbers=(((1,), (0,)), ((), ())),
        preferred_element_type=jnp.float32,
    )
    h = h + b_ref[...]
    part = pl.program_id(0) % _WGROUP
    o_ref[:, pl.ds(part * _BLOCK_C, _BLOCK_C)] = (
        jnp.floor(h * (1.0 / _NUM_BUCKETS)).astype(jnp.int32))


@jax.jit
def kernel(x, projections, biases):
    n, emb = x.shape
    num_hashes = projections.shape[0]
    xt = x.T
    grid = (pl.cdiv(n, _BLOCK_C),)
    out_t = pl.pallas_call(
        _lsh_block_kernel,
        grid=grid,
        in_specs=[
            pl.BlockSpec((emb, _BLOCK_C), lambda i: (0, i)),
            pl.BlockSpec((num_hashes, emb), lambda i: (0, 0)),
            pl.BlockSpec((num_hashes, 1), lambda i: (0, 0)),
        ],
        out_specs=pl.BlockSpec((num_hashes, _WGROUP * _BLOCK_C),
                               lambda i: (0, i // _WGROUP)),
        out_shape=jax.ShapeDtypeStruct((num_hashes, n), jnp.int32),
    )(xt, projections, biases.reshape(num_hashes, 1))
    return out_t.T


# speedup vs baseline: 1.0051x; 1.0051x over previous
import jax
import jax.numpy as jnp
from jax.experimental import pallas as pl

_NUM_BUCKETS = 1024.0
_BLOCK_C = 49152


def _lsh_block_kernel(xt_ref, p_ref, b_ref, o_ref):
    h = jax.lax.dot_general(
        p_ref[...], xt_ref[...],
        dimension_numbers=(((1,), (0,)), ((), ())),
        preferred_element_type=jnp.float32,
    )
    h = h + b_ref[...]
    o_ref[...] = jnp.floor(h * (1.0 / _NUM_BUCKETS)).astype(jnp.int32)


@jax.jit
def kernel(x, projections, biases):
    n, emb = x.shape
    num_hashes = projections.shape[0]
    xt = x.T
    grid = (pl.cdiv(n, _BLOCK_C),)
    out_t = pl.pallas_call(
        _lsh_block_kernel,
        grid=grid,
        in_specs=[
            pl.BlockSpec((emb, _BLOCK_C), lambda i: (0, i)),
            pl.BlockSpec((num_hashes, emb), lambda i: (0, 0)),
            pl.BlockSpec((num_hashes, 1), lambda i: (0, 0)),
        ],
        out_specs=pl.BlockSpec((num_hashes, _BLOCK_C), lambda i: (0, i)),
        out_shape=jax.ShapeDtypeStruct((num_hashes, n), jnp.int32),
    )(xt, projections, biases.reshape(num_hashes, 1))
    return out_t.T
